# speedup vs baseline: 2.6476x; 2.6476x over previous
import functools

import jax
import jax.numpy as jnp
from jax import lax
from jax.experimental import pallas as pl
from jax.experimental.pallas import tpu as pltpu

_N = 10000
_E = 320000
_NPAD = 10240
_BR = 512
_BC = 1024


def _decode_body(mu_ref, muc_ref, out_ref):
    a = mu_ref[...]
    b = muc_ref[...]
    v = lax.dot_general(a, b, (((1,), (1,)), ((), ())),
                        preferred_element_type=jnp.float32)
    out_ref[...] = jax.nn.sigmoid(v)


def _decode(mu_pad):
    grid = (_NPAD // _BR, _NPAD // _BC)
    return pl.pallas_call(
        _decode_body,
        grid=grid,
        in_specs=[
            pl.BlockSpec((_BR, 16), lambda i, j: (i, 0)),
            pl.BlockSpec((_BC, 16), lambda i, j: (j, 0)),
        ],
        out_specs=pl.BlockSpec((_BR, _BC), lambda i, j: (i, j)),
        out_shape=jax.ShapeDtypeStruct((_N, _N), jnp.float32),
        compiler_params=pltpu.CompilerParams(
            dimension_semantics=("parallel", "parallel")),
    )(mu_pad, mu_pad)


def kernel(x, edge_index, W_gc1, b_gc1, Wm_root, Wm_nbr, bm, Wv_root, Wv_nbr, bv):
    n = x.shape[0]
    src = edge_index[0]
    dst = edge_index[1]

    deg = jnp.zeros((n,), x.dtype).at[dst].add(1.0) + 1.0
    dinv = lax.rsqrt(deg)
    h = x @ W_gc1
    g = h * dinv[:, None]
    agg1 = jnp.zeros((n, 64), x.dtype).at[dst].add(g[src])
    h1 = dinv[:, None] * (agg1 + g) + b_gc1
    agg2 = jnp.zeros((n, 64), x.dtype).at[dst].add(h1[src])
    Wr = jnp.concatenate([Wm_root, Wv_root], axis=1)
    Wn = jnp.concatenate([Wm_nbr, Wv_nbr], axis=1)
    bc = jnp.concatenate([bm, bv])
    out32 = h1 @ Wr + agg2 @ Wn + bc
    mu = out32[:, :16]
    logvar = out32[:, 16:]

    mu_pad = jnp.pad(mu, ((0, _NPAD - n), (0, 0)))
    adj = _decode(mu_pad)
    return (adj, mu, mu, logvar)
